# baseline (speedup 1.0000x reference)
import functools
import jax
import jax.numpy as jnp
from jax.experimental import pallas as pl
from jax.experimental.pallas import tpu as pltpu

N = 10000
C = 1250
NB = 1000


def _embed_kernel(fea_ref, w_ref, embW_ref, embb_ref, out_ref):
    x = jnp.dot(fea_ref[...], embW_ref[...], preferred_element_type=jnp.float32)
    x = x + embb_ref[...]
    out_ref[...] = jnp.concatenate([x, w_ref[...]], axis=1)


def _embed(elem_fea, elem_weights, embW, embb):
    return pl.pallas_call(
        _embed_kernel,
        grid=(N // NB,),
        in_specs=[
            pl.BlockSpec((NB, 128), lambda i: (i, 0)),
            pl.BlockSpec((NB, 1), lambda i: (i, 0)),
            pl.BlockSpec((128, 63), lambda i: (0, 0)),
            pl.BlockSpec((1, 63), lambda i: (0, 0)),
        ],
        out_specs=pl.BlockSpec((NB, 64), lambda i: (i, 0)),
        out_shape=jax.ShapeDtypeStruct((N, 64), jnp.float32),
    )(elem_fea, elem_weights, embW, embb.reshape(1, 63))


def _wap_decomposed(x, w, lw, Ag, Bg, wg2, Am, Bm, Wm2, bm2, idx_self, idx_nbr,
                    nseg):
    hg = jax.nn.leaky_relu(Ag[idx_self] + Bg[idx_nbr], 0.01)
    g = hg @ wg2 + lw[idx_nbr]
    m = jax.ops.segment_max(g, idx_self, num_segments=nseg)
    p = jnp.exp(g - m[idx_self])
    Z = jax.ops.segment_sum(p, idx_self, num_segments=nseg)
    hm = jax.nn.leaky_relu(Am[idx_self] + Bm[idx_nbr], 0.01)
    P = jax.ops.segment_sum(p[:, None] * hm, idx_self, num_segments=nseg)
    Zs = Z / (Z + 1e-10)
    P = P / (Z + 1e-10)[:, None]
    return P @ Wm2 + Zs[:, None] * bm2


def kernel(elem_weights, elem_fea, self_fea_idx, nbr_fea_idx, cry_elem_idx, params):
    x = _embed(elem_fea, elem_weights, params["embW"], params["embb"])
    w = elem_weights[:, 0]
    logw = jnp.log(w)

    for gp in params["graphs"]:
        outs = []
        for hp in gp["heads"]:
            Wg1, bg1 = hp["gate"]["hidden"][0]
            wg2 = hp["gate"]["outW"][:, 0]
            Wm1, bm1 = hp["message"]["hidden"][0]
            Wm2, bm2 = hp["message"]["outW"], hp["message"]["outb"]
            pw = hp["pow"][0]
            Ag = x @ Wg1[:64] + bg1
            Bg = x @ Wg1[64:]
            Am = x @ Wm1[:64] + bm1
            Bm = x @ Wm1[64:]
            lw = jnp.where(pw > 0, pw * logw,
                           -jnp.log(jnp.exp(jnp.abs(pw) * logw) + 1e-10))
            outs.append(_wap_decomposed(x, w, lw, Ag, Bg, wg2, Am, Bm, Wm2, bm2,
                                        self_fea_idx, nbr_fea_idx, N))
        x = jnp.mean(jnp.stack(outs), axis=0) + x

    outs = []
    ninds = jnp.arange(N, dtype=jnp.int32)
    for hp in params["cry"]:
        Wg1, bg1 = hp["gate"]["hidden"][0]
        wg2 = hp["gate"]["outW"][:, 0]
        Wm1, bm1 = hp["message"]["hidden"][0]
        Wm2, bm2 = hp["message"]["outW"], hp["message"]["outb"]
        pw = hp["pow"][0]
        lw = jnp.where(pw > 0, pw * logw,
                       -jnp.log(jnp.exp(jnp.abs(pw) * logw) + 1e-10))
        g = jax.nn.leaky_relu(x @ Wg1 + bg1, 0.01) @ wg2 + lw
        hm = jax.nn.leaky_relu(x @ Wm1 + bm1, 0.01)
        m = jax.ops.segment_max(g, cry_elem_idx, num_segments=C)
        p = jnp.exp(g - m[cry_elem_idx])
        Z = jax.ops.segment_sum(p, cry_elem_idx, num_segments=C)
        P = jax.ops.segment_sum(p[:, None] * hm, cry_elem_idx, num_segments=C)
        Zs = Z / (Z + 1e-10)
        P = P / (Z + 1e-10)[:, None]
        outs.append(P @ Wm2 + Zs[:, None] * bm2)
    return jnp.mean(jnp.stack(outs), axis=0)


# speedup vs baseline: 5.6996x; 5.6996x over previous
import functools
import jax
import jax.numpy as jnp
from jax import lax
from jax.experimental import pallas as pl
from jax.experimental.pallas import tpu as pltpu
from jax.experimental.pallas import tpu_sc as plsc

N = 10000
M = 320000
C = 1250
NB = 1000
NW = 32
EPW = M // NW
K = 64
BROW = 896
RROW = 400
K2 = 16
NCH2 = N // K2

@functools.cache
def _mesh():
    return plsc.VectorSubcoreMesh(core_axis_name="c", subcore_axis_name="s")


_GD = lax.GatherDimensionNumbers(
    offset_dims=(), collapsed_slice_dims=(0,), start_index_map=(0,))


def _permute(v, idx):
    return lax.gather(v, idx[:, None], _GD, (1,),
                      mode=lax.GatherScatterMode.PROMISE_IN_BOUNDS)


def _hsum(v):
    for d in (8, 4, 2, 1):
        v = v + _permute(v, lax.iota(jnp.int32, 16) ^ d)
    return v


def _blane(v, lane):
    return _permute(v, jnp.full((16,), lane, jnp.int32))


def _full(x):
    return jnp.full((16,), x, jnp.float32)


def _embed_body(fea_ref, w_ref, embW_ref, embb_ref, out_ref):
    x = jnp.dot(fea_ref[...], embW_ref[...], preferred_element_type=jnp.float32)
    out_ref[...] = jnp.concatenate([x + embb_ref[...], w_ref[...]], axis=1)


def _embed(elem_fea, elem_weights, embW, embb):
    return pl.pallas_call(
        _embed_body,
        grid=(N // NB,),
        in_specs=[
            pl.BlockSpec((NB, 128), lambda i: (i, 0)),
            pl.BlockSpec((NB, 1), lambda i: (i, 0)),
            pl.BlockSpec((128, 63), lambda i: (0, 0)),
            pl.BlockSpec((1, 63), lambda i: (0, 0)),
        ],
        out_specs=pl.BlockSpec((NB, 64), lambda i: (i, 0)),
        out_shape=jax.ShapeDtypeStruct((N, 64), jnp.float32),
    )(elem_fea, elem_weights, embW, embb.reshape(1, 63))


def _tables_body(x_ref, w_ref, WA_ref, bA_ref, WB_ref, pw_ref, A_ref, B_ref):
    x = x_ref[...]
    A_ref[...] = jnp.dot(x, WA_ref[...], preferred_element_type=jnp.float32) + bA_ref[...]
    Bm = jnp.dot(x, WB_ref[...], preferred_element_type=jnp.float32)
    logw = jnp.log(w_ref[...])
    cols = [Bm]
    for h in range(3):
        pw = pw_ref[0, h]
        lw = jnp.where(pw > 0, pw * logw,
                       -jnp.log(jnp.exp(jnp.abs(pw) * logw) + 1e-10))
        cols.append(lw)
    cols.append(jnp.zeros((x.shape[0], BROW - 771), jnp.float32))
    B_ref[...] = jnp.concatenate(cols, axis=1)


def _tables(x, w, WA, bA, WB, pw):
    return pl.pallas_call(
        _tables_body,
        grid=(N // NB,),
        in_specs=[
            pl.BlockSpec((NB, 64), lambda i: (i, 0)),
            pl.BlockSpec((NB, 1), lambda i: (i, 0)),
            pl.BlockSpec((64, 768), lambda i: (0, 0)),
            pl.BlockSpec((1, 768), lambda i: (0, 0)),
            pl.BlockSpec((64, 768), lambda i: (0, 0)),
            pl.BlockSpec((1, 8), lambda i: (0, 0)),
        ],
        out_specs=[
            pl.BlockSpec((NB, 768), lambda i: (i, 0)),
            pl.BlockSpec((NB, BROW), lambda i: (i, 0)),
        ],
        out_shape=[
            jax.ShapeDtypeStruct((N, 768), jnp.float32),
            jax.ShapeDtypeStruct((N, BROW), jnp.float32),
        ],
    )(x, w, WA, bA, WB, pw)


def _combine_body(P_ref, zs_ref, x_ref, W2_ref, B2_ref, out_ref):
    out_ref[...] = (jnp.dot(P_ref[...], W2_ref[...], preferred_element_type=jnp.float32)
                    + jnp.dot(zs_ref[...], B2_ref[...], preferred_element_type=jnp.float32)
                    ) * (1.0 / 3.0) + x_ref[...]


def _combine(P, zs, x, W2, B2):
    return pl.pallas_call(
        _combine_body,
        grid=(N // NB,),
        in_specs=[
            pl.BlockSpec((NB, 384), lambda i: (i, 0)),
            pl.BlockSpec((NB, 16), lambda i: (i, 0)),
            pl.BlockSpec((NB, 64), lambda i: (i, 0)),
            pl.BlockSpec((384, 64), lambda i: (0, 0)),
            pl.BlockSpec((16, 64), lambda i: (0, 0)),
        ],
        out_specs=pl.BlockSpec((NB, 64), lambda i: (i, 0)),
        out_shape=jax.ShapeDtypeStruct((N, 64), jnp.float32),
    )(P, zs, x, W2, B2)


def _crydense_body(x_ref, w_ref, Wg_ref, bg_ref, wg2_ref, Wm_ref, bm_ref,
                   pw_ref, R_ref):
    x = x_ref[...]
    G = jnp.dot(x, Wg_ref[...], preferred_element_type=jnp.float32) + bg_ref[...]
    G = jnp.maximum(G, 0.01 * G)
    g3 = jnp.dot(G, wg2_ref[...], preferred_element_type=jnp.float32)
    Hm = jnp.dot(x, Wm_ref[...], preferred_element_type=jnp.float32) + bm_ref[...]
    Hm = jnp.maximum(Hm, 0.01 * Hm)
    logw = jnp.log(w_ref[...])
    cols = [Hm]
    for h in range(3):
        pw = pw_ref[0, h]
        lw = jnp.where(pw > 0, pw * logw,
                       -jnp.log(jnp.exp(jnp.abs(pw) * logw) + 1e-10))
        cols.append(g3[:, h:h + 1] + lw)
    cols.append(jnp.zeros((x.shape[0], RROW - 387), jnp.float32))
    R_ref[...] = jnp.concatenate(cols, axis=1)


def _crydense(x, w, Wg, bg, wg2blk, Wm, bm, pw):
    return pl.pallas_call(
        _crydense_body,
        grid=(N // NB,),
        in_specs=[
            pl.BlockSpec((NB, 64), lambda i: (i, 0)),
            pl.BlockSpec((NB, 1), lambda i: (i, 0)),
            pl.BlockSpec((64, 384), lambda i: (0, 0)),
            pl.BlockSpec((1, 384), lambda i: (0, 0)),
            pl.BlockSpec((384, 8), lambda i: (0, 0)),
            pl.BlockSpec((64, 384), lambda i: (0, 0)),
            pl.BlockSpec((1, 384), lambda i: (0, 0)),
            pl.BlockSpec((1, 8), lambda i: (0, 0)),
        ],
        out_specs=pl.BlockSpec((NB, RROW), lambda i: (i, 0)),
        out_shape=jax.ShapeDtypeStruct((N, RROW), jnp.float32),
    )(x, w, Wg, bg, wg2blk, Wm, bm, pw)


def _crycomb_body(P_ref, zs_ref, W2_ref, B2_ref, out_ref):
    out_ref[...] = (jnp.dot(P_ref[...], W2_ref[...], preferred_element_type=jnp.float32)
                    + jnp.dot(zs_ref[...], B2_ref[...], preferred_element_type=jnp.float32)
                    ) * (1.0 / 3.0)


def _crycomb(P, zs, W2, B2):
    return pl.pallas_call(
        _crycomb_body,
        in_specs=[pl.BlockSpec((C, 384), lambda: (0, 0)),
                  pl.BlockSpec((C, 16), lambda: (0, 0)),
                  pl.BlockSpec((384, 64), lambda: (0, 0)),
                  pl.BlockSpec((16, 64), lambda: (0, 0))],
        out_specs=pl.BlockSpec((C, 64), lambda: (0, 0)),
        out_shape=jax.ShapeDtypeStruct((C, 64), jnp.float32),
    )(P, zs, W2, B2)


def _zero_rows(lo, hi, zrow, z16, P_hbm, zs_hbm):
    @pl.loop(lo, hi)
    def _z(i):
        pltpu.sync_copy(zrow, P_hbm.at[pl.ds(i, 1)])
        pltpu.sync_copy(z16, zs_hbm.at[pl.ds(i, 1)])


def _reset_state(mzbuf, accbuf):
    z = jnp.zeros((16,), jnp.float32)
    for h in range(3):
        mzbuf[h, :] = _full(-1e30)
        mzbuf[3 + h, :] = z
    for i in range(24):
        accbuf[i, :] = z


def _finalize(node, mzbuf, accbuf, p_buf, zs_buf, P_hbm, zs_hbm):
    zsv = jnp.zeros((16,), jnp.float32)
    for h in range(3):
        Z = mzbuf[3 + h, :]
        inv = 1.0 / (Z + 1e-10)
        for k in range(8):
            p_buf[0, pl.ds((h * 8 + k) * 16, 16)] = accbuf[h * 8 + k, :] * inv
        oh = lax.iota(jnp.int32, 16) == h
        zsv = jnp.where(oh, Z * inv, zsv)
    zs_buf[0, :] = zsv
    pltpu.sync_copy(p_buf, P_hbm.at[pl.ds(node, 1)])
    pltpu.sync_copy(zs_buf, zs_hbm.at[pl.ds(node, 1)])


def _seg_step(s, e, range_end, prevnode, st, mzbuf, accbuf, p_buf, zs_buf,
              zrow, z16, P_hbm, zs_hbm, on_begin):
    cur, last, done = st
    newseg = jnp.logical_and(done == 0, s != cur)
    fin = jnp.logical_and(newseg, cur >= 0)

    @pl.when(fin)
    def _():
        _finalize(cur, mzbuf, accbuf, p_buf, zs_buf, P_hbm, zs_hbm)
    last = jnp.where(fin, cur, last)

    oor = jnp.logical_and(newseg, e >= range_end)
    startp = jnp.logical_and(newseg, e < range_end)
    beginp = jnp.logical_and(startp, s != prevnode)

    @pl.when(startp)
    def _():
        _zero_rows(last + 1, s, zrow, z16, P_hbm, zs_hbm)

    @pl.when(beginp)
    def _():
        on_begin()
        _reset_state(mzbuf, accbuf)

    newcur = jnp.where(newseg,
                       jnp.where(beginp, s, jnp.int32(-1)), cur)
    newdone = jnp.where(oor, jnp.int32(1), done)
    return (newcur, last, newdone)


def _edge_kernel_body(A_hbm, B_hbm, self_hbm, nbr_hbm, wg2_hbm,
                      P_hbm, zs_hbm,
                      sbuf, nbuf, bbuf, abuf, wgbuf, p_buf, zs_buf,
                      zrow, z16, pbuf16, mzbuf, accbuf, stb, sem):
    w = lax.axis_index("s") * 2 + lax.axis_index("c")
    range_end = (w + 1) * EPW

    pltpu.sync_copy(wg2_hbm, wgbuf)
    for i in range(24):
        zrow[0, pl.ds(i * 16, 16)] = jnp.zeros((16,), jnp.float32)
    z16[0, :] = jnp.zeros((16,), jnp.float32)

    @pl.when(w > 0)
    def _():
        pltpu.sync_copy(self_hbm.at[pl.ds(w * EPW - 16, 16)], pbuf16)
    prevnode = jnp.where(w > 0, pbuf16[...][15], jnp.int32(-1))

    stb[0] = jnp.int32(-1)
    stb[1] = prevnode
    stb[2] = jnp.int32(0)

    fzero = jnp.zeros((16,), jnp.float32)
    nchunks = (M - w * EPW + K - 1) // K

    pltpu.sync_copy(self_hbm.at[pl.ds(w * EPW, K)], sbuf.at[0, pl.ds(0, K)])
    pltpu.sync_copy(nbr_hbm.at[pl.ds(w * EPW, K)], nbuf.at[0])
    pltpu.async_copy(B_hbm.at[nbuf.at[0]], bbuf.at[0], sem.at[0])
    stb[3] = jnp.int32(1)
    stb[4] = jnp.int32(0)

    @pl.loop(0, nchunks)
    def _chunk(ci):
        base = w * EPW + ci * K

        @pl.when(stb[2] == 0)
        def _():
            par = lax.rem(ci, 2)

            @pl.when(ci + 1 < nchunks)
            def _():
                par2 = lax.rem(ci + 1, 2)
                pltpu.sync_copy(self_hbm.at[pl.ds(base + K, K)],
                                sbuf.at[par2, pl.ds(0, K)])
                pltpu.sync_copy(nbr_hbm.at[pl.ds(base + K, K)], nbuf.at[par2])
                pltpu.async_copy(B_hbm.at[nbuf.at[par2]], bbuf.at[par2],
                                 sem.at[par2])
                stb[3] = stb[3] + 1

            pltpu.make_async_copy(B_hbm.at[nbuf.at[par]], bbuf.at[par],
                                  sem.at[par]).wait()
            stb[4] = stb[4] + 1

            def edge_body(j, st2):
                s = sbuf[par, pl.ds(j, 16)][0]

                def on_begin():
                    pltpu.sync_copy(A_hbm.at[pl.ds(s, 1)], abuf)
                cur, last2, done2 = _seg_step(
                    s, base + j, range_end, prevnode, st2,
                    mzbuf, accbuf, p_buf, zs_buf, zrow, z16,
                    P_hbm, zs_hbm, on_begin)

                @pl.when(cur >= 0)
                def _():
                    for h in range(3):
                        part = fzero
                        for k in range(8):
                            off = h * 256 + k * 16
                            hg = abuf[0, pl.ds(off, 16)] + bbuf[par, j, pl.ds(off, 16)]
                            hg = jnp.maximum(hg, 0.01 * hg)
                            part = part + hg * wgbuf[h, pl.ds(k * 16, 16)]
                        tail = bbuf[par, j, pl.ds(768, 16)]
                        oh = lax.iota(jnp.int32, 16) == h
                        part = part + jnp.where(oh, tail, fzero)
                        g = _hsum(part)
                        m = mzbuf[h, :]
                        Z = mzbuf[3 + h, :]
                        newm = jnp.maximum(m, g)
                        r = jnp.exp(m - newm)
                        p = jnp.exp(g - newm)
                        mzbuf[h, :] = newm
                        mzbuf[3 + h, :] = Z * r + p
                        for k in range(8):
                            off = h * 256 + 128 + k * 16
                            hm = abuf[0, pl.ds(off, 16)] + bbuf[par, j, pl.ds(off, 16)]
                            hm = jnp.maximum(hm, 0.01 * hm)
                            accbuf[h * 8 + k, :] = accbuf[h * 8 + k, :] * r + p * hm
                return (cur, last2, done2)

            st2 = lax.fori_loop(0, K, edge_body,
                                (stb[0], stb[1], stb[2]))
            stb[0] = st2[0]
            stb[1] = st2[1]
            stb[2] = st2[2]

    @pl.when(stb[3] > stb[4])
    def _():
        p = lax.rem(stb[4], 2)
        pltpu.make_async_copy(B_hbm.at[nbuf.at[p]], bbuf.at[p],
                              sem.at[p]).wait()

    cur = stb[0]

    @pl.when(cur >= 0)
    def _():
        _finalize(cur, mzbuf, accbuf, p_buf, zs_buf, P_hbm, zs_hbm)
    last = jnp.where(cur >= 0, cur, stb[1])

    @pl.when(w == NW - 1)
    def _():
        _zero_rows(last + 1, jnp.int32(N), zrow, z16, P_hbm, zs_hbm)


@functools.cache
def _mk_edge_kernel():
  return pl.kernel(
    _edge_kernel_body,
    out_type=(jax.ShapeDtypeStruct((N, 384), jnp.float32),
              jax.ShapeDtypeStruct((N, 16), jnp.float32)),
    mesh=_mesh(),
    scratch_types=[
        pltpu.VMEM((2, K + 16), jnp.int32),
        pltpu.VMEM((2, K), jnp.int32),
        pltpu.VMEM((2, K, BROW), jnp.float32),
        pltpu.VMEM((1, 768), jnp.float32),
        pltpu.VMEM((3, 128), jnp.float32),
        pltpu.VMEM((1, 384), jnp.float32),
        pltpu.VMEM((1, 16), jnp.float32),
        pltpu.VMEM((1, 384), jnp.float32),
        pltpu.VMEM((1, 16), jnp.float32),
        pltpu.VMEM((16,), jnp.int32),
        pltpu.VMEM((8, 16), jnp.float32),
        pltpu.VMEM((24, 16), jnp.float32),
        pltpu.SMEM((8,), jnp.int32),
        pltpu.SemaphoreType.DMA((2,)),
    ],
  )


def _cry_kernel_body(R_hbm, cidx_hbm, Pc_hbm, zsc_hbm,
                     ibuf, rbuf, p_buf, zs_buf, zrow, z16, pbuf16,
                     mzbuf, accbuf, stb):
    w = lax.axis_index("s") * 2 + lax.axis_index("c")
    c_lo = (w * NCH2) // NW
    c_hi = ((w + 1) * NCH2) // NW
    range_end = c_hi * K2

    for i in range(24):
        zrow[0, pl.ds(i * 16, 16)] = jnp.zeros((16,), jnp.float32)
    z16[0, :] = jnp.zeros((16,), jnp.float32)

    @pl.when(w > 0)
    def _():
        pltpu.sync_copy(cidx_hbm.at[pl.ds(c_lo * K2 - 16, 16)], pbuf16)
    prevnode = jnp.where(w > 0, pbuf16[...][15], jnp.int32(-1))

    stb[0] = jnp.int32(-1)
    stb[1] = prevnode
    stb[2] = jnp.int32(0)

    @pl.loop(c_lo, NCH2)
    def _chunk(ci):
        base = ci * K2

        @pl.when(stb[2] == 0)
        def _():
            pltpu.sync_copy(cidx_hbm.at[pl.ds(base, K2)], ibuf.at[pl.ds(0, K2)])
            pltpu.sync_copy(R_hbm.at[pl.ds(base, K2)], rbuf)

            def item_body(j, st2):
                s = ibuf[pl.ds(j, 16)][0]
                cur, last2, done2 = _seg_step(
                    s, base + j, range_end, prevnode, st2,
                    mzbuf, accbuf, p_buf, zs_buf, zrow, z16,
                    Pc_hbm, zsc_hbm, lambda: None)

                @pl.when(cur >= 0)
                def _():
                    tail = rbuf[j, pl.ds(384, 16)]
                    for h in range(3):
                        g = _blane(tail, h)
                        m = mzbuf[h, :]
                        Z = mzbuf[3 + h, :]
                        newm = jnp.maximum(m, g)
                        r = jnp.exp(m - newm)
                        p = jnp.exp(g - newm)
                        mzbuf[h, :] = newm
                        mzbuf[3 + h, :] = Z * r + p
                        for k in range(8):
                            hm = rbuf[j, pl.ds(h * 128 + k * 16, 16)]
                            accbuf[h * 8 + k, :] = accbuf[h * 8 + k, :] * r + p * hm
                return (cur, last2, done2)

            st2 = lax.fori_loop(0, K2, item_body,
                                (stb[0], stb[1], stb[2]))
            stb[0] = st2[0]
            stb[1] = st2[1]
            stb[2] = st2[2]

    cur = stb[0]

    @pl.when(cur >= 0)
    def _():
        _finalize(cur, mzbuf, accbuf, p_buf, zs_buf, Pc_hbm, zsc_hbm)
    last = jnp.where(cur >= 0, cur, stb[1])

    @pl.when(w == NW - 1)
    def _():
        _zero_rows(last + 1, jnp.int32(C), zrow, z16, Pc_hbm, zsc_hbm)


@functools.cache
def _mk_cry_kernel():
  return pl.kernel(
    _cry_kernel_body,
    out_type=(jax.ShapeDtypeStruct((C, 384), jnp.float32),
              jax.ShapeDtypeStruct((C, 16), jnp.float32)),
    mesh=_mesh(),
    scratch_types=[
        pltpu.VMEM((K2 + 16,), jnp.int32),
        pltpu.VMEM((K2, RROW), jnp.float32),
        pltpu.VMEM((1, 384), jnp.float32),
        pltpu.VMEM((1, 16), jnp.float32),
        pltpu.VMEM((1, 384), jnp.float32),
        pltpu.VMEM((1, 16), jnp.float32),
        pltpu.VMEM((16,), jnp.int32),
        pltpu.VMEM((8, 16), jnp.float32),
        pltpu.VMEM((24, 16), jnp.float32),
        pltpu.SMEM((8,), jnp.int32),
    ],
  )


def kernel(elem_weights, elem_fea, self_fea_idx, nbr_fea_idx, cry_elem_idx, params):
    x = _embed(elem_fea, elem_weights, params["embW"], params["embb"])
    w = elem_weights
    zpad = jnp.zeros((K,), jnp.int32)
    self_pad = jnp.concatenate([self_fea_idx, zpad])
    nbr_pad = jnp.concatenate([nbr_fea_idx, zpad])

    for gp in params["graphs"]:
        WA_cols, bA_cols, WB_cols, wg2_rows, pws = [], [], [], [], []
        W2_rows, B2_rows = [], []
        for hp in gp["heads"]:
            Wg1, bg1 = hp["gate"]["hidden"][0]
            Wm1, bm1 = hp["message"]["hidden"][0]
            WA_cols += [Wg1[:64], Wm1[:64]]
            bA_cols += [bg1, bm1]
            WB_cols += [Wg1[64:], Wm1[64:]]
            wg2_rows.append(hp["gate"]["outW"][:, 0])
            pws.append(hp["pow"][0])
            W2_rows.append(hp["message"]["outW"])
            B2_rows.append(hp["message"]["outb"])
        WA = jnp.concatenate(WA_cols, axis=1)
        bA = jnp.concatenate(bA_cols).reshape(1, 768)
        WB = jnp.concatenate(WB_cols, axis=1)
        wg2 = jnp.stack(wg2_rows)
        pw = jnp.stack(pws + [jnp.float32(0)] * 5).reshape(1, 8)
        W2 = jnp.concatenate(W2_rows, axis=0)
        B2 = jnp.concatenate(
            [jnp.stack(B2_rows), jnp.zeros((13, 64), jnp.float32)], axis=0)

        A, B = _tables(x, w, WA, bA, WB, pw)
        P, zs = _mk_edge_kernel()(A, B, self_pad, nbr_pad, wg2)
        x = _combine(P, zs, x, W2, B2)

    Wg_cols, bg_cols, Wm_cols, bm_cols, pws = [], [], [], [], []
    W2_rows, B2_rows = [], []
    wg2blk = jnp.zeros((384, 8), jnp.float32)
    for h, hp in enumerate(params["cry"]):
        Wg1, bg1 = hp["gate"]["hidden"][0]
        Wm1, bm1 = hp["message"]["hidden"][0]
        Wg_cols.append(Wg1)
        bg_cols.append(bg1)
        Wm_cols.append(Wm1)
        bm_cols.append(bm1)
        wg2blk = wg2blk.at[h * 128:(h + 1) * 128, h].set(hp["gate"]["outW"][:, 0])
        pws.append(hp["pow"][0])
        W2_rows.append(hp["message"]["outW"])
        B2_rows.append(hp["message"]["outb"])
    Wg = jnp.concatenate(Wg_cols, axis=1)
    bg = jnp.concatenate(bg_cols).reshape(1, 384)
    Wm = jnp.concatenate(Wm_cols, axis=1)
    bm = jnp.concatenate(bm_cols).reshape(1, 384)
    pw = jnp.stack(pws + [jnp.float32(0)] * 5).reshape(1, 8)
    W2c = jnp.concatenate(W2_rows, axis=0)
    B2c = jnp.concatenate(
        [jnp.stack(B2_rows), jnp.zeros((13, 64), jnp.float32)], axis=0)

    R = _crydense(x, w, Wg, bg, wg2blk, Wm, bm, pw)
    Pc, zsc = _mk_cry_kernel()(R, cry_elem_idx)
    return _crycomb(Pc, zsc, W2c, B2c)
